# Initial kernel scaffold; baseline (speedup 1.0000x reference)
import jax
import jax.numpy as jnp
from jax.experimental import pallas as pl


def kernel(state, action, W1, b1, Wd, bd, Wc, bc, Wa1, ba1, Wa2, ba2):
    raise NotImplementedError("write your pallas kernel here")


# speedup vs baseline: 1.2330x; 1.2330x over previous
import functools
import math

import jax
import jax.numpy as jnp
import numpy as np
from jax.experimental import pallas as pl

_ACTION_SIZES = (5, 2, 4, 3, 2, 9, 2, 32, 35, 7, 2, 21, 2, 3, 3)
_NSEG = len(_ACTION_SIZES)
_DISC = 132
_BB = 1024
_HALF_LOG_2PI = 0.5 * math.log(2.0 * math.pi)


def _np_constants():
    S = np.zeros((_DISC, _NSEG), dtype=np.float32)
    starts = np.cumsum([0] + list(_ACTION_SIZES))
    for s, (c0, c1) in enumerate(zip(starts[:-1], starts[1:])):
        S[c0:c1, s] = 1.0
    E_cont = np.zeros((155, 23), dtype=np.float32)
    for i in range(23):
        E_cont[132 + i, i] = 1.0
    E_agent = np.zeros((161, 6), dtype=np.float32)
    for i in range(6):
        E_agent[155 + i, i] = 1.0
    return S, E_cont, E_agent, starts


_S_NP, _ECONT_NP, _EAGENT_NP, _STARTS = _np_constants()


def _policy_kernel(state_ref, action_ref, w1t_ref, b1_ref, wdt_ref, bd_ref,
                   wcmt_ref, bcm_ref, wcst_ref, bcs_ref, wa1t_ref, ba1_ref,
                   wamt_ref, bam_ref, wast_ref, bas_ref, s_ref, econt_ref,
                   eagent_ref, out_ref):
    f32 = jnp.float32
    x = state_ref[...]
    act = action_ref[...]

    h = jnp.dot(x, w1t_ref[...], preferred_element_type=f32) + b1_ref[...]
    h = jnp.where(h >= 0.0, h, 0.01 * h)

    logits = jnp.dot(h, wdt_ref[...], preferred_element_type=f32) + bd_ref[...]
    mean = jnp.clip(jnp.dot(h, wcmt_ref[...], preferred_element_type=f32)
                    + bcm_ref[...], -1.0, 1.0)
    logstd = jnp.clip(jnp.dot(h, wcst_ref[...], preferred_element_type=f32)
                      + bcs_ref[...], 0.0, 1.0)

    disc = act[:, :_DISC]
    continuous = jnp.dot(act, econt_ref[...], preferred_element_type=f32)

    gmax = jnp.max(logits, axis=1, keepdims=True)
    e = jnp.exp(logits - gmax)
    segsum = jnp.dot(e, s_ref[...], preferred_element_type=f32)
    lse = jnp.log(segsum) + gmax

    iota = jax.lax.broadcasted_iota(jnp.int32, logits.shape, 1)
    oh = jnp.zeros(logits.shape, dtype=f32)
    for s in range(_NSEG):
        c0, c1 = int(_STARTS[s]), int(_STARTS[s + 1])
        inseg = (iota >= c0) & (iota < c1)
        dm = jnp.max(jnp.where(inseg, disc, -jnp.inf), axis=1, keepdims=True)
        cand = jnp.where(inseg & (disc == dm), iota, _DISC)
        idx = jnp.min(cand, axis=1, keepdims=True)
        oh = oh + (iota == idx).astype(f32)
    chosen = jnp.dot(logits * oh, s_ref[...], preferred_element_type=f32)
    seg_lp = chosen - lse

    cont_lp = (-(continuous - mean) ** 2 * (0.5 * jnp.exp(-2.0 * logstd))
               - logstd - _HALF_LOG_2PI)

    h2 = jnp.dot(x, wa1t_ref[...], preferred_element_type=f32) + ba1_ref[...]
    h2 = jnp.where(h2 >= 0.0, h2, 0.01 * h2)
    m2 = jnp.clip(jnp.dot(h2, wamt_ref[...], preferred_element_type=f32)
                  + bam_ref[...], -1.0, 1.0)
    ls2 = jnp.clip(jnp.dot(h2, wast_ref[...], preferred_element_type=f32)
                   + bas_ref[...], 0.0, 1.0)
    aact = jnp.dot(x, eagent_ref[...], preferred_element_type=f32)
    agent_lp = (-(aact - m2) ** 2 * (0.5 * jnp.exp(-2.0 * ls2))
                - ls2 - _HALF_LOG_2PI)

    out_ref[...] = jnp.concatenate([seg_lp, cont_lp, agent_lp], axis=1)


@functools.partial(jax.jit, static_argnames=("interpret",))
def _run(state, action, W1, b1, Wd, bd, Wc, bc, Wa1, ba1, Wa2, ba2,
         interpret=False):
    B = state.shape[0]
    w1t = W1.T
    bd2 = bd[None, :]
    wdt = Wd.T
    wcmt = Wc[:23].T
    bcm = bc[None, :23]
    wcst = Wc[23:].T
    bcs = bc[None, 23:]
    wa1t = jnp.zeros((161, 128), jnp.float32).at[:155, :].set(Wa1.T)
    wamt = Wa2[:6].T
    bam = ba2[None, :6]
    wast = Wa2[6:].T
    bas = ba2[None, 6:]
    S = jnp.asarray(_S_NP)
    econt = jnp.asarray(_ECONT_NP)
    eagent = jnp.asarray(_EAGENT_NP)

    grid = (B // _BB,)
    row = lambda i: (i, 0)
    rep = lambda i: (0, 0)
    full = lambda a: pl.BlockSpec(a.shape, rep)
    out = pl.pallas_call(
        _policy_kernel,
        grid=grid,
        in_specs=[
            pl.BlockSpec((_BB, 161), row),
            pl.BlockSpec((_BB, 155), row),
            full(w1t), full(b1[None, :]), full(wdt), full(bd2),
            full(wcmt), full(bcm), full(wcst), full(bcs),
            full(wa1t), full(ba1[None, :]),
            full(wamt), full(bam), full(wast), full(bas),
            full(S), full(econt), full(eagent),
        ],
        out_specs=pl.BlockSpec((_BB, 44), row),
        out_shape=jax.ShapeDtypeStruct((B, 44), jnp.float32),
        interpret=interpret,
    )(state, action, w1t, b1[None, :], wdt, bd2, wcmt, bcm, wcst, bcs,
      wa1t, ba1[None, :], wamt, bam, wast, bas, S, econt, eagent)
    return out


def kernel(state, action, W1, b1, Wd, bd, Wc, bc, Wa1, ba1, Wa2, ba2):
    return _run(state, action, W1, b1, Wd, bd, Wc, bc, Wa1, ba1, Wa2, ba2)


# speedup vs baseline: 1.3190x; 1.0697x over previous
import functools
import math

import jax
import jax.numpy as jnp
import numpy as np
from jax.experimental import pallas as pl

_ACTION_SIZES = (5, 2, 4, 3, 2, 9, 2, 32, 35, 7, 2, 21, 2, 3, 3)
_NSEG = len(_ACTION_SIZES)
_DISC = 132
_BB = 1024
_HALF_LOG_2PI = 0.5 * math.log(2.0 * math.pi)


def _np_constants():
    S = np.zeros((_DISC, _NSEG), dtype=np.float32)
    starts = np.cumsum([0] + list(_ACTION_SIZES))
    for s, (c0, c1) in enumerate(zip(starts[:-1], starts[1:])):
        S[c0:c1, s] = 1.0
    E_cont = np.zeros((155, 23), dtype=np.float32)
    for i in range(23):
        E_cont[132 + i, i] = 1.0
    E_agent = np.zeros((161, 6), dtype=np.float32)
    for i in range(6):
        E_agent[155 + i, i] = 1.0
    return S, E_cont, E_agent, starts


_S_NP, _ECONT_NP, _EAGENT_NP, _STARTS = _np_constants()


def _policy_kernel(state_ref, action_ref, w1t_ref, b1_ref, wdt_ref, bd_ref,
                   wcmt_ref, bcm_ref, wcst_ref, bcs_ref, wa1t_ref, ba1_ref,
                   wamt_ref, bam_ref, wast_ref, bas_ref, s_ref, econt_ref,
                   eagent_ref, out_ref):
    f32 = jnp.float32
    x = state_ref[...]
    act = action_ref[...]

    h = jnp.dot(x, w1t_ref[...], preferred_element_type=f32) + b1_ref[...]
    h = jnp.where(h >= 0.0, h, 0.01 * h)

    logits = jnp.dot(h, wdt_ref[...], preferred_element_type=f32) + bd_ref[...]
    mean = jnp.clip(jnp.dot(h, wcmt_ref[...], preferred_element_type=f32)
                    + bcm_ref[...], -1.0, 1.0)
    logstd = jnp.clip(jnp.dot(h, wcst_ref[...], preferred_element_type=f32)
                      + bcs_ref[...], 0.0, 1.0)

    disc = act[:, :_DISC]
    continuous = jnp.dot(act, econt_ref[...], preferred_element_type=f32)

    gmax = jnp.max(logits, axis=1, keepdims=True)
    e = jnp.exp(logits - gmax)
    segsum = jnp.dot(e, s_ref[...], preferred_element_type=f32)
    lse = jnp.log(segsum) + gmax

    iota = jax.lax.broadcasted_iota(jnp.int32, logits.shape, 1)
    oh = jnp.zeros(logits.shape, dtype=f32)
    for s in range(_NSEG):
        c0, c1 = int(_STARTS[s]), int(_STARTS[s + 1])
        idx = jnp.argmax(disc[:, c0:c1], axis=1)[:, None] + c0
        oh = oh + (iota == idx).astype(f32)
    chosen = jnp.dot(logits * oh, s_ref[...], preferred_element_type=f32)
    seg_lp = chosen - lse

    cont_lp = (-(continuous - mean) ** 2 * (0.5 * jnp.exp(-2.0 * logstd))
               - logstd - _HALF_LOG_2PI)

    h2 = jnp.dot(x, wa1t_ref[...], preferred_element_type=f32) + ba1_ref[...]
    h2 = jnp.where(h2 >= 0.0, h2, 0.01 * h2)
    m2 = jnp.clip(jnp.dot(h2, wamt_ref[...], preferred_element_type=f32)
                  + bam_ref[...], -1.0, 1.0)
    ls2 = jnp.clip(jnp.dot(h2, wast_ref[...], preferred_element_type=f32)
                   + bas_ref[...], 0.0, 1.0)
    aact = jnp.dot(x, eagent_ref[...], preferred_element_type=f32)
    agent_lp = (-(aact - m2) ** 2 * (0.5 * jnp.exp(-2.0 * ls2))
                - ls2 - _HALF_LOG_2PI)

    out_ref[...] = jnp.concatenate([seg_lp, cont_lp, agent_lp], axis=1)


@functools.partial(jax.jit, static_argnames=("interpret",))
def _run(state, action, W1, b1, Wd, bd, Wc, bc, Wa1, ba1, Wa2, ba2,
         interpret=False):
    B = state.shape[0]
    w1t = W1.T
    bd2 = bd[None, :]
    wdt = Wd.T
    wcmt = Wc[:23].T
    bcm = bc[None, :23]
    wcst = Wc[23:].T
    bcs = bc[None, 23:]
    wa1t = jnp.zeros((161, 128), jnp.float32).at[:155, :].set(Wa1.T)
    wamt = Wa2[:6].T
    bam = ba2[None, :6]
    wast = Wa2[6:].T
    bas = ba2[None, 6:]
    S = jnp.asarray(_S_NP)
    econt = jnp.asarray(_ECONT_NP)
    eagent = jnp.asarray(_EAGENT_NP)

    grid = (B // _BB,)
    row = lambda i: (i, 0)
    rep = lambda i: (0, 0)
    full = lambda a: pl.BlockSpec(a.shape, rep)
    out = pl.pallas_call(
        _policy_kernel,
        grid=grid,
        in_specs=[
            pl.BlockSpec((_BB, 161), row),
            pl.BlockSpec((_BB, 155), row),
            full(w1t), full(b1[None, :]), full(wdt), full(bd2),
            full(wcmt), full(bcm), full(wcst), full(bcs),
            full(wa1t), full(ba1[None, :]),
            full(wamt), full(bam), full(wast), full(bas),
            full(S), full(econt), full(eagent),
        ],
        out_specs=pl.BlockSpec((_BB, 44), row),
        out_shape=jax.ShapeDtypeStruct((B, 44), jnp.float32),
        interpret=interpret,
    )(state, action, w1t, b1[None, :], wdt, bd2, wcmt, bcm, wcst, bcs,
      wa1t, ba1[None, :], wamt, bam, wast, bas, S, econt, eagent)
    return out


def kernel(state, action, W1, b1, Wd, bd, Wc, bc, Wa1, ba1, Wa2, ba2):
    return _run(state, action, W1, b1, Wd, bd, Wc, bc, Wa1, ba1, Wa2, ba2)
